# Initial kernel scaffold; baseline (speedup 1.0000x reference)
import jax
import jax.numpy as jnp
from jax.experimental import pallas as pl


def kernel(ui_index, ui_values, i2i_index, i2i_values, u2u_index, u2u_values, user_ui_w, item_ui_w, uu_w, ii_w):
    raise NotImplementedError("write your pallas kernel here")


# speedup vs baseline: 2.8471x; 2.8471x over previous
import functools

import jax
import jax.numpy as jnp
from jax import lax
from jax.experimental import pallas as pl
from jax.experimental.pallas import tpu as pltpu
from jax.experimental.pallas import tpu_sc as plsc

_N_USERS = 25000
_N_ITEMS = 25000
_D = 64
_HALF = 25000
_RPT = 1568
_ACC_ROWS = 16 * _RPT
_C = 128
_NT = 16
_L = 16
_PAD_ROW = 1 << 20


def _make_spmm(e_sc: int, per_sc_split: bool, row_off_unit: int, col_off_unit: int):
    ept = e_sc // _NT
    chunks = ept // _C
    mesh = plsc.VectorSubcoreMesh(core_axis_name="c", subcore_axis_name="s")

    def body(row_h, col_h, val_h, x_h, zeros_h, out_h, acc, idxr, idxc, vals, rows, sem):
        cid = lax.axis_index("c")
        sid = lax.axis_index("s")
        row_off = cid * row_off_unit
        col_off = cid * col_off_unit
        pltpu.sync_copy(zeros_h, acc.at[pl.ds(sid * _RPT, _RPT)])
        plsc.subcore_barrier()
        base0 = sid * ept + (cid * e_sc if per_sc_split else 0)

        def chunk(ci, carry):
            base = base0 + ci * _C
            pltpu.sync_copy(row_h.at[pl.ds(base, _C)], idxr)
            pltpu.sync_copy(col_h.at[pl.ds(base, _C)], idxc)
            pltpu.sync_copy(val_h.at[pl.ds(base, _C)], vals)
            for g in range(_C // _L):
                sl = pl.ds(g * _L, _L)
                r16 = idxr[sl] - row_off
                c16 = idxc[sl]
                oob = (r16 < 0) | (r16 >= _HALF)
                idxr[sl] = jnp.where(oob, _HALF + (c16 & 63), r16)
                if col_off_unit:
                    idxc[sl] = c16 + col_off
            pltpu.async_copy(x_h.at[idxc], rows, sem).wait()
            for g in range(_C // _L):
                v16 = vals[pl.ds(g * _L, _L)]
                for e in range(_L):
                    s = lax.squeeze(lax.slice(v16, (e,), (e + 1,)), dimensions=(0,))
                    vb = jnp.broadcast_to(s, (_L,))
                    r = g * _L + e
                    for k in range(_D // _L):
                        csl = pl.ds(k * _L, _L)
                        rows[r, csl] = rows[r, csl] * vb
            pltpu.sync_copy(rows, acc.at[idxr], add=True)
            return carry

        lax.fori_loop(0, chunks, chunk, 0)
        plsc.subcore_barrier()
        pltpu.sync_copy(acc.at[pl.ds(sid * _RPT, _RPT)],
                        out_h.at[cid, pl.ds(sid * _RPT, _RPT)])

    return pl.kernel(
        body,
        mesh=mesh,
        out_type=jax.ShapeDtypeStruct((2, _ACC_ROWS, _D), jnp.float32),
        scratch_types=[
            pltpu.VMEM_SHARED((_ACC_ROWS, _D), jnp.float32),
            pltpu.VMEM((_C,), jnp.int32),
            pltpu.VMEM((_C,), jnp.int32),
            pltpu.VMEM((_C,), jnp.float32),
            pltpu.VMEM((_C, _D), jnp.float32),
            pltpu.SemaphoreType.DMA,
        ],
        compiler_params=pltpu.CompilerParams(use_tc_tiling_on_sc=False),
    )


_E_UI_PAD = 800768
_E_G_PAD = 401408

_spmm_ui = _make_spmm(_E_UI_PAD, per_sc_split=False, row_off_unit=_HALF, col_off_unit=0)
_spmm_cmb = _make_spmm(_E_G_PAD, per_sc_split=True, row_off_unit=0, col_off_unit=_N_ITEMS)


def _pad_to(a, n, fill):
    p = n - a.shape[0]
    if p == 0:
        return a
    return jnp.concatenate([a, jnp.full((p,), fill, a.dtype)])


_B = 1000


def _epi_body(uw, iw, e1u, e1i, e2u, e2i, uu, ii, uo, io):
    def l2(x):
        n = jnp.sqrt(jnp.sum(x * x, axis=1, keepdims=True))
        return x / jnp.maximum(n, 1e-12)

    uo[...] = (uw[...] + e1u[...] + e2u[...]) / 3.0 + l2(uu[...])
    io[...] = (iw[...] + e1i[...] + e2i[...]) / 3.0 + l2(ii[...])


def _epilogue(uw, iw, e1, e2, uu, ii):
    nb = _N_USERS // _B

    def ix(i):
        return (i, 0)

    def ix_item(i):
        return (i + nb, 0)

    bs = pl.BlockSpec((_B, _D), ix)
    bs_item = pl.BlockSpec((_B, _D), ix_item)
    return pl.pallas_call(
        _epi_body,
        grid=(nb,),
        in_specs=[bs, bs, bs, bs_item, bs, bs_item, bs, bs],
        out_specs=[bs, bs],
        out_shape=[
            jax.ShapeDtypeStruct((_N_USERS, _D), jnp.float32),
            jax.ShapeDtypeStruct((_N_ITEMS, _D), jnp.float32),
        ],
    )(uw, iw, e1, e1, e2, e2, uu, ii)


def kernel(ui_index, ui_values, i2i_index, i2i_values, u2u_index, u2u_values,
           user_ui_w, item_ui_w, uu_w, ii_w):
    zeros = jnp.zeros((_RPT, _D), jnp.float32)

    rows_c = jnp.concatenate([
        _pad_to(i2i_index[0], _E_G_PAD, _PAD_ROW),
        _pad_to(u2u_index[0], _E_G_PAD, _PAD_ROW),
    ])
    cols_c = jnp.concatenate([
        _pad_to(i2i_index[1], _E_G_PAD, 0),
        _pad_to(u2u_index[1], _E_G_PAD, 0),
    ])
    vals_c = jnp.concatenate([
        _pad_to(i2i_values, _E_G_PAD, 0.0),
        _pad_to(u2u_values, _E_G_PAD, 0.0),
    ])
    x_c = jnp.concatenate([ii_w, uu_w], axis=0)
    out_c = _spmm_cmb(rows_c, cols_c, vals_c, x_c, zeros)
    ii_emb = out_c[0, :_N_ITEMS]
    uu_emb = out_c[1, :_N_USERS]

    rows_ui = _pad_to(ui_index[0], _E_UI_PAD, _PAD_ROW)
    cols_ui = _pad_to(ui_index[1], _E_UI_PAD, 0)
    vals_ui = _pad_to(ui_values, _E_UI_PAD, 0.0)
    ego0 = jnp.concatenate([user_ui_w, item_ui_w], axis=0)
    o1 = _spmm_ui(rows_ui, cols_ui, vals_ui, ego0, zeros)
    e1 = jnp.concatenate([o1[0, :_HALF], o1[1, :_HALF]], axis=0)
    o2 = _spmm_ui(rows_ui, cols_ui, vals_ui, e1, zeros)
    e2 = jnp.concatenate([o2[0, :_HALF], o2[1, :_HALF]], axis=0)

    u_ui_emb, i_ui_emb = _epilogue(user_ui_w, item_ui_w, e1, e2, uu_emb, ii_emb)
    return (u_ui_emb, i_ui_emb, ii_emb, uu_emb)


# speedup vs baseline: 3.8807x; 1.3630x over previous
import functools

import jax
import jax.numpy as jnp
from jax import lax
from jax.experimental import pallas as pl
from jax.experimental.pallas import tpu as pltpu
from jax.experimental.pallas import tpu_sc as plsc

_N_USERS = 25000
_N_ITEMS = 25000
_D = 64
_HALF = 25000
_RPT = 1568
_ACC_ROWS = 16 * _RPT
_C = 80
_NT = 16
_L = 16
_PAD_ROW = 1 << 20
_NB = 5
_DP = 3
_PKW = 2 * _C


def _make_spmm(e_sc: int, per_sc_split: bool, row_off_unit: int, col_off_unit: int):
    ept = e_sc // _NT
    chunks = ept // _C
    assert chunks % _NB == 0 and chunks >= _NB
    mesh = plsc.VectorSubcoreMesh(core_axis_name="c", subcore_axis_name="s")

    def body(pk_h, pv_h, x_h, zeros_h, out_h, acc, pk, vals, idr, idc, rows, sg, ss):
        cid = lax.axis_index("c")
        sid = lax.axis_index("s")
        row_off = cid * row_off_unit
        col_off = cid * col_off_unit
        pltpu.sync_copy(zeros_h, acc.at[pl.ds(sid * _RPT, _RPT)])
        plsc.subcore_barrier()
        ci0 = sid * chunks + (cid * _NT * chunks if per_sc_split else 0)

        def prep(ci, b):
            pltpu.sync_copy(pk_h.at[pl.ds(ci * _PKW, _PKW)], pk.at[b])
            pltpu.sync_copy(pv_h.at[pl.ds(ci * _C, _C)], vals.at[b])
            for g in range(_C // _L):
                sl = pl.ds(g * _L, _L)
                r16 = pk[b, sl] - row_off
                c16 = pk[b, pl.ds(_C + g * _L, _L)]
                oob = (r16 < 0) | (r16 >= _HALF)
                idr.at[b][sl] = jnp.where(oob, _HALF + (c16 & 63), r16)
                idc.at[b][sl] = c16 + col_off
            pltpu.async_copy(x_h.at[idc.at[b]], rows.at[b], sg.at[b])

        def wait_g(b):
            pltpu.make_async_copy(x_h.at[idc.at[b]], rows.at[b], sg.at[b]).wait()

        def start_s(b):
            pltpu.async_copy(rows.at[b], acc.at[idr.at[b]], ss.at[b], add=True)

        def wait_s(b):
            pltpu.make_async_copy(rows.at[b], acc.at[idr.at[b]], ss.at[b]).wait()

        def scale(b):
            rv = rows.at[b]
            for g in range(_C // _L):
                v16 = vals[b, pl.ds(g * _L, _L)]
                for e in range(_L):
                    s = lax.squeeze(lax.slice(v16, (e,), (e + 1,)), dimensions=(0,))
                    vb = jnp.broadcast_to(s, (_L,))
                    r = g * _L + e
                    for k in range(_D // _L):
                        csl = pl.ds(k * _L, _L)
                        rv[r, csl] = rv[r, csl] * vb

        for b in range(_DP):
            prep(ci0 + b, b)

        def round_body(j, carry):
            for b in range(_NB):
                wait_g(b)
                scale(b)
                start_s(b)
                pt = _NB * j + b + _DP
                pb = (b + _DP) % _NB

                @pl.when(pt < chunks)
                def _():
                    if b < _NB - _DP:
                        @pl.when(j > 0)
                        def _():
                            wait_s(pb)
                    else:
                        wait_s(pb)
                    prep(ci0 + pt, pb)

            return carry

        lax.fori_loop(0, chunks // _NB, round_body, 0)
        for b in range(_NB):
            wait_s(b)
        plsc.subcore_barrier()
        pltpu.sync_copy(acc.at[pl.ds(sid * _RPT, _RPT)],
                        out_h.at[cid, pl.ds(sid * _RPT, _RPT)])

    return pl.kernel(
        body,
        mesh=mesh,
        out_type=jax.ShapeDtypeStruct((2, _ACC_ROWS, _D), jnp.float32),
        scratch_types=[
            pltpu.VMEM_SHARED((_ACC_ROWS, _D), jnp.float32),
            pltpu.VMEM((_NB, _PKW), jnp.int32),
            pltpu.VMEM((_NB, _C), jnp.float32),
            pltpu.VMEM((_NB, _C), jnp.int32),
            pltpu.VMEM((_NB, _C), jnp.int32),
            pltpu.VMEM((_NB, _C, _D), jnp.float32),
            pltpu.SemaphoreType.DMA((_NB,)),
            pltpu.SemaphoreType.DMA((_NB,)),
        ],
        compiler_params=pltpu.CompilerParams(use_tc_tiling_on_sc=False),
    )


_E_UI_PAD = 800000
_E_G_PAD = 403200

_spmm_ui = _make_spmm(_E_UI_PAD, per_sc_split=False, row_off_unit=_HALF, col_off_unit=0)
_spmm_cmb = _make_spmm(_E_G_PAD, per_sc_split=True, row_off_unit=0, col_off_unit=_N_ITEMS)


def _pack_edges(rows, cols, vals, n_pad):
    def pad1(a, fill):
        p = n_pad - a.shape[0]
        return jnp.concatenate([a, jnp.full((p,), fill, a.dtype)]) if p else a

    r = pad1(rows, _PAD_ROW).reshape(-1, _C)
    c = pad1(cols, 0).reshape(-1, _C)
    return jnp.stack([r, c], axis=1).reshape(-1), pad1(vals, 0.0)


_B = 1000


def _epi_body(uw, iw, e1u, e1i, e2u, e2i, uu, ii, uo, io):
    def l2(x):
        n = jnp.sqrt(jnp.sum(x * x, axis=1, keepdims=True))
        return x / jnp.maximum(n, 1e-12)

    uo[...] = (uw[...] + e1u[...] + e2u[...]) / 3.0 + l2(uu[...])
    io[...] = (iw[...] + e1i[...] + e2i[...]) / 3.0 + l2(ii[...])


def _epilogue(uw, iw, e1, e2, uu, ii):
    nb = _N_USERS // _B

    def ix(i):
        return (i, 0)

    def ix_item(i):
        return (i + nb, 0)

    bs = pl.BlockSpec((_B, _D), ix)
    bs_item = pl.BlockSpec((_B, _D), ix_item)
    return pl.pallas_call(
        _epi_body,
        grid=(nb,),
        in_specs=[bs, bs, bs, bs_item, bs, bs_item, bs, bs],
        out_specs=[bs, bs],
        out_shape=[
            jax.ShapeDtypeStruct((_N_USERS, _D), jnp.float32),
            jax.ShapeDtypeStruct((_N_ITEMS, _D), jnp.float32),
        ],
    )(uw, iw, e1, e1, e2, e2, uu, ii)


def kernel(ui_index, ui_values, i2i_index, i2i_values, u2u_index, u2u_values,
           user_ui_w, item_ui_w, uu_w, ii_w):
    zeros = jnp.zeros((_RPT, _D), jnp.float32)

    pk_i, pv_i = _pack_edges(i2i_index[0], i2i_index[1], i2i_values, _E_G_PAD)
    pk_u, pv_u = _pack_edges(u2u_index[0], u2u_index[1], u2u_values, _E_G_PAD)
    pk_c = jnp.concatenate([pk_i, pk_u])
    pv_c = jnp.concatenate([pv_i, pv_u])
    x_c = jnp.concatenate([ii_w, uu_w], axis=0)
    out_c = _spmm_cmb(pk_c, pv_c, x_c, zeros)
    ii_emb = out_c[0, :_N_ITEMS]
    uu_emb = out_c[1, :_N_USERS]

    pk_ui, pv_ui = _pack_edges(ui_index[0], ui_index[1], ui_values, _E_UI_PAD)
    ego0 = jnp.concatenate([user_ui_w, item_ui_w], axis=0)
    o1 = _spmm_ui(pk_ui, pv_ui, ego0, zeros)
    e1 = jnp.concatenate([o1[0, :_HALF], o1[1, :_HALF]], axis=0)
    o2 = _spmm_ui(pk_ui, pv_ui, e1, zeros)
    e2 = jnp.concatenate([o2[0, :_HALF], o2[1, :_HALF]], axis=0)

    u_ui_emb, i_ui_emb = _epilogue(user_ui_w, item_ui_w, e1, e2, uu_emb, ii_emb)
    return (u_ui_emb, i_ui_emb, ii_emb, uu_emb)


# speedup vs baseline: 7.3544x; 1.8951x over previous
import functools

import jax
import jax.numpy as jnp
from jax import lax
from jax.experimental import pallas as pl
from jax.experimental.pallas import tpu as pltpu
from jax.experimental.pallas import tpu_sc as plsc

_N_USERS = 25000
_N_ITEMS = 25000
_D = 64
_N_JOINT = _N_USERS + _N_ITEMS
_NT = 16
_L = 16
_NB = 5
_DPL = 4
_DPG = 2


def _make_spmm(e_sc: int, per_sc_split: bool, col_off_unit: int,
               d: int, acc_rows: int, c: int):
    ept = e_sc // _NT
    chunks = ept // c
    assert chunks % _NB == 0 and chunks >= _NB
    rpt = acc_rows // _NT
    pkw = 2 * c
    mesh = plsc.VectorSubcoreMesh(core_axis_name="c", subcore_axis_name="s")

    def body(pk_h, pv_h, x_h, zeros_h, out_h, acc, pk, vals, idr, idc, rows,
             sp, sg, ss):
        cid = lax.axis_index("c")
        sid = lax.axis_index("s")
        col_off = cid * col_off_unit
        pltpu.sync_copy(zeros_h, acc.at[pl.ds(sid * rpt, rpt)])
        plsc.subcore_barrier()
        ci0 = sid * chunks + (cid * _NT * chunks if per_sc_split else 0)

        def load(ci, b):
            pltpu.async_copy(pk_h.at[pl.ds(ci * pkw, pkw)], pk.at[b], sp.at[b])
            pltpu.async_copy(pv_h.at[pl.ds(ci * c, c)], vals.at[b], sp.at[b])

        def gather_prep(ci, b):
            pltpu.make_async_copy(pk_h.at[pl.ds(ci * pkw, pkw)], pk.at[b],
                                  sp.at[b]).wait()
            pltpu.make_async_copy(pv_h.at[pl.ds(ci * c, c)], vals.at[b],
                                  sp.at[b]).wait()
            for g in range(c // _L):
                sl = pl.ds(g * _L, _L)
                idr.at[b][sl] = pk[b, sl]
                idc.at[b][sl] = pk[b, pl.ds(c + g * _L, _L)] + col_off
            pltpu.async_copy(x_h.at[idc.at[b]], rows.at[b], sg.at[b])

        def wait_g(b):
            pltpu.make_async_copy(x_h.at[idc.at[b]], rows.at[b], sg.at[b]).wait()

        def start_s(b):
            pltpu.async_copy(rows.at[b], acc.at[idr.at[b]], ss.at[b], add=True)

        def wait_s(b):
            pltpu.make_async_copy(rows.at[b], acc.at[idr.at[b]], ss.at[b]).wait()

        def scale(b):
            rv = rows.at[b]
            for g in range(c // _L):
                v16 = vals[b, pl.ds(g * _L, _L)]
                for e in range(_L):
                    s = lax.squeeze(lax.slice(v16, (e,), (e + 1,)), dimensions=(0,))
                    vb = jnp.broadcast_to(s, (_L,))
                    r = g * _L + e
                    for k in range(d // _L):
                        csl = pl.ds(k * _L, _L)
                        rv[r, csl] = rv[r, csl] * vb

        for b in range(_DPL):
            load(ci0 + b, b)
        for b in range(_DPG):
            gather_prep(ci0 + b, b)

        def round_body(j, carry):
            for b in range(_NB):
                ii = _NB * j + b
                wait_g(b)
                scale(b)
                start_s(b)

                lt = ii + _DPL

                @pl.when(lt < chunks)
                def _():
                    load(ci0 + lt, (b + _DPL) % _NB)

                gt = ii + _DPG
                bg = (b + _DPG) % _NB

                @pl.when(gt < chunks)
                def _():
                    if b < _NB - _DPG:
                        @pl.when(j > 0)
                        def _():
                            wait_s(bg)
                    else:
                        wait_s(bg)
                    gather_prep(ci0 + gt, bg)

            return carry

        lax.fori_loop(0, chunks // _NB, round_body, 0)
        for b in range(_NB):
            wait_s(b)
        plsc.subcore_barrier()
        pltpu.sync_copy(acc.at[pl.ds(sid * rpt, rpt)],
                        out_h.at[cid, pl.ds(sid * rpt, rpt)])

    return pl.kernel(
        body,
        mesh=mesh,
        out_type=jax.ShapeDtypeStruct((2, acc_rows, d), jnp.float32),
        scratch_types=[
            pltpu.VMEM_SHARED((acc_rows, d), jnp.float32),
            pltpu.VMEM((_NB, pkw), jnp.int32),
            pltpu.VMEM((_NB, c), jnp.float32),
            pltpu.VMEM((_NB, c), jnp.int32),
            pltpu.VMEM((_NB, c), jnp.int32),
            pltpu.VMEM((_NB, c, d), jnp.float32),
            pltpu.SemaphoreType.DMA((_NB,)),
            pltpu.SemaphoreType.DMA((_NB,)),
            pltpu.SemaphoreType.DMA((_NB,)),
        ],
        compiler_params=pltpu.CompilerParams(use_tc_tiling_on_sc=False),
    )


_C_UI = 128
_C_G = 80
_E_UI_PAD = 808960
_E_G_PAD = 403200
_ACC_UI = 50176
_ACC_G = 25088

_spmm_ui = _make_spmm(_E_UI_PAD, per_sc_split=False, col_off_unit=_N_JOINT,
                      d=32, acc_rows=_ACC_UI, c=_C_UI)
_spmm_cmb = _make_spmm(_E_G_PAD, per_sc_split=True, col_off_unit=_N_ITEMS,
                       d=_D, acc_rows=_ACC_G, c=_C_G)


def _pack_edges(rows, cols, vals, n_pad, c, pad_row):
    def pad1(a, fill):
        p = n_pad - a.shape[0]
        return jnp.concatenate([a, jnp.full((p,), fill, a.dtype)]) if p else a

    r = pad1(rows, pad_row).reshape(-1, c)
    cc = pad1(cols, 0).reshape(-1, c)
    return jnp.stack([r, cc], axis=1).reshape(-1), pad1(vals, 0.0)


_B = 1000


def _epi_body(uw, iw, e1u, e1i, e2u, e2i, uu, ii, uo, io):
    def l2(x):
        n = jnp.sqrt(jnp.sum(x * x, axis=1, keepdims=True))
        return x / jnp.maximum(n, 1e-12)

    uo[...] = (uw[...] + e1u[...] + e2u[...]) / 3.0 + l2(uu[...])
    io[...] = (iw[...] + e1i[...] + e2i[...]) / 3.0 + l2(ii[...])


def _epilogue(uw, iw, e1, e2, uu, ii):
    nb = _N_USERS // _B

    def ix(i):
        return (i, 0)

    def ix_item(i):
        return (i + nb, 0)

    bs = pl.BlockSpec((_B, _D), ix)
    bs_item = pl.BlockSpec((_B, _D), ix_item)
    return pl.pallas_call(
        _epi_body,
        grid=(nb,),
        in_specs=[bs, bs, bs, bs_item, bs, bs_item, bs, bs],
        out_specs=[bs, bs],
        out_shape=[
            jax.ShapeDtypeStruct((_N_USERS, _D), jnp.float32),
            jax.ShapeDtypeStruct((_N_ITEMS, _D), jnp.float32),
        ],
    )(uw, iw, e1, e1, e2, e2, uu, ii)


def kernel(ui_index, ui_values, i2i_index, i2i_values, u2u_index, u2u_values,
           user_ui_w, item_ui_w, uu_w, ii_w):
    zeros_ui = jnp.zeros((_ACC_UI // _NT, 32), jnp.float32)
    zeros_g = jnp.zeros((_ACC_G // _NT, _D), jnp.float32)

    pk_i, pv_i = _pack_edges(i2i_index[0], i2i_index[1], i2i_values,
                             _E_G_PAD, _C_G, _N_ITEMS)
    pk_u, pv_u = _pack_edges(u2u_index[0], u2u_index[1], u2u_values,
                             _E_G_PAD, _C_G, _N_USERS)
    pk_c = jnp.concatenate([pk_i, pk_u])
    pv_c = jnp.concatenate([pv_i, pv_u])
    x_c = jnp.concatenate([ii_w, uu_w], axis=0)
    out_c = _spmm_cmb(pk_c, pv_c, x_c, zeros_g)
    ii_emb = out_c[0, :_N_ITEMS]
    uu_emb = out_c[1, :_N_USERS]

    pk_ui, pv_ui = _pack_edges(ui_index[0], ui_index[1], ui_values,
                               _E_UI_PAD, _C_UI, _N_JOINT)
    ego0 = jnp.concatenate([user_ui_w, item_ui_w], axis=0)
    xs0 = jnp.concatenate([ego0[:, :32], ego0[:, 32:]], axis=0)
    o1 = _spmm_ui(pk_ui, pv_ui, xs0, zeros_ui)
    xs1 = jnp.concatenate([o1[0, :_N_JOINT], o1[1, :_N_JOINT]], axis=0)
    o2 = _spmm_ui(pk_ui, pv_ui, xs1, zeros_ui)
    e1 = jnp.concatenate([o1[0, :_N_JOINT], o1[1, :_N_JOINT]], axis=1)
    e2 = jnp.concatenate([o2[0, :_N_JOINT], o2[1, :_N_JOINT]], axis=1)

    u_ui_emb, i_ui_emb = _epilogue(user_ui_w, item_ui_w, e1, e2, uu_emb, ii_emb)
    return (u_ui_emb, i_ui_emb, ii_emb, uu_emb)


# speedup vs baseline: 7.7542x; 1.0544x over previous
import functools

import jax
import jax.numpy as jnp
from jax import lax
from jax.experimental import pallas as pl
from jax.experimental.pallas import tpu as pltpu
from jax.experimental.pallas import tpu_sc as plsc

_N_USERS = 25000
_N_ITEMS = 25000
_D = 64
_HD = 32
_N_JOINT = _N_USERS + _N_ITEMS
_NT = 16
_L = 16
_NB = 5
_DPL = 4
_DPG = 2


def _make_spmm(e_sc: int, per_sc_split: bool, col_off_unit: int,
               d: int, acc_rows: int, c: int):
    ept = e_sc // _NT
    chunks = ept // c
    assert chunks % _NB == 0 and chunks >= _NB
    rpt = acc_rows // _NT
    mesh = plsc.VectorSubcoreMesh(core_axis_name="c", subcore_axis_name="s")

    def body(row_h, col_h, val_h, x_h, zeros_h, out_h, acc, pk, vals, idr, idc,
             rows, sp, sg, ss):
        cid = lax.axis_index("c")
        sid = lax.axis_index("s")
        col_off = cid * col_off_unit
        ci0 = sid * chunks + (cid * _NT * chunks if per_sc_split else 0)

        def load(ci, b):
            e0 = ci * c
            pltpu.async_copy(row_h.at[pl.ds(e0, c)], pk.at[b, 0], sp.at[b])
            pltpu.async_copy(col_h.at[pl.ds(e0, c)], pk.at[b, 1], sp.at[b])
            pltpu.async_copy(val_h.at[pl.ds(e0, c)], vals.at[b], sp.at[b])

        def gather_prep(ci, b):
            e0 = ci * c
            pltpu.make_async_copy(row_h.at[pl.ds(e0, c)], pk.at[b, 0], sp.at[b]).wait()
            pltpu.make_async_copy(col_h.at[pl.ds(e0, c)], pk.at[b, 1], sp.at[b]).wait()
            pltpu.make_async_copy(val_h.at[pl.ds(e0, c)], vals.at[b], sp.at[b]).wait()
            for g in range(c // _L):
                sl = pl.ds(g * _L, _L)
                idr.at[b][sl] = pk[b, 0, sl]
                idc.at[b][sl] = pk[b, 1, sl] + col_off
            pltpu.async_copy(x_h.at[idc.at[b]], rows.at[b], sg.at[b])

        def wait_g(b):
            pltpu.make_async_copy(x_h.at[idc.at[b]], rows.at[b], sg.at[b]).wait()

        def start_s(b):
            pltpu.async_copy(rows.at[b], acc.at[idr.at[b]], ss.at[b], add=True)

        def wait_s(b):
            pltpu.make_async_copy(rows.at[b], acc.at[idr.at[b]], ss.at[b]).wait()

        def scale(b):
            rv = rows.at[b]
            for g in range(c // _L):
                v16 = vals[b, pl.ds(g * _L, _L)]
                for e in range(_L):
                    s = lax.squeeze(lax.slice(v16, (e,), (e + 1,)), dimensions=(0,))
                    vb = jnp.broadcast_to(s, (_L,))
                    r = g * _L + e
                    for k in range(d // _L):
                        csl = pl.ds(k * _L, _L)
                        rv[r, csl] = rv[r, csl] * vb

        for b in range(_DPL):
            load(ci0 + b, b)
        pltpu.sync_copy(zeros_h, acc.at[pl.ds(sid * rpt, rpt)])
        for b in range(_DPG):
            gather_prep(ci0 + b, b)
        plsc.subcore_barrier()

        def round_body(j, carry):
            for b in range(_NB):
                ii = _NB * j + b
                wait_g(b)
                scale(b)
                start_s(b)

                lt = ii + _DPL

                @pl.when(lt < chunks)
                def _():
                    load(ci0 + lt, (b + _DPL) % _NB)

                gt = ii + _DPG
                bg = (b + _DPG) % _NB

                @pl.when(gt < chunks)
                def _():
                    if b < _NB - _DPG:
                        @pl.when(j > 0)
                        def _():
                            wait_s(bg)
                    else:
                        wait_s(bg)
                    gather_prep(ci0 + gt, bg)

            return carry

        lax.fori_loop(0, chunks // _NB, round_body, 0)
        for b in range(_NB):
            wait_s(b)
        plsc.subcore_barrier()
        pltpu.sync_copy(acc.at[pl.ds(sid * rpt, rpt)],
                        out_h.at[cid, pl.ds(sid * rpt, rpt)])

    return pl.kernel(
        body,
        mesh=mesh,
        out_type=jax.ShapeDtypeStruct((2, acc_rows, d), jnp.float32),
        scratch_types=[
            pltpu.VMEM_SHARED((acc_rows, d), jnp.float32),
            pltpu.VMEM((_NB, 2, c), jnp.int32),
            pltpu.VMEM((_NB, c), jnp.float32),
            pltpu.VMEM((_NB, c), jnp.int32),
            pltpu.VMEM((_NB, c), jnp.int32),
            pltpu.VMEM((_NB, c, d), jnp.float32),
            pltpu.SemaphoreType.DMA((_NB,)),
            pltpu.SemaphoreType.DMA((_NB,)),
            pltpu.SemaphoreType.DMA((_NB,)),
        ],
        compiler_params=pltpu.CompilerParams(use_tc_tiling_on_sc=False),
    )


_C_UI = 128
_C_G = 80
_E_UI_PAD = 808960
_E_G_PAD = 403200
_ACC_UI = _N_JOINT
_ACC_G = 25088

_spmm_ui = _make_spmm(_E_UI_PAD, per_sc_split=False, col_off_unit=_N_JOINT,
                      d=_HD, acc_rows=_ACC_UI, c=_C_UI)
_spmm_cmb = _make_spmm(_E_G_PAD, per_sc_split=True, col_off_unit=_N_ITEMS,
                       d=_D, acc_rows=_ACC_G, c=_C_G)


def _pad1(a, n_pad, fill):
    p = n_pad - a.shape[0]
    return jnp.concatenate([a, jnp.full((p,), fill, a.dtype)]) if p else a


_B = 1000


def _epi_body(uw, iw, e1ul, e1uh, e1il, e1ih, e2ul, e2uh, e2il, e2ih,
              uu, ii, uo, io):
    def l2(x):
        n = jnp.sqrt(jnp.sum(x * x, axis=1, keepdims=True))
        return x / jnp.maximum(n, 1e-12)

    e1u = jnp.concatenate([e1ul[0], e1uh[0]], axis=1)
    e1i = jnp.concatenate([e1il[0], e1ih[0]], axis=1)
    e2u = jnp.concatenate([e2ul[0], e2uh[0]], axis=1)
    e2i = jnp.concatenate([e2il[0], e2ih[0]], axis=1)
    uo[...] = (uw[...] + e1u + e2u) / 3.0 + l2(uu[...])
    io[...] = (iw[...] + e1i + e2i) / 3.0 + l2(ii[...])


def _epilogue(uw, iw, o1, o2, uu, ii):
    nb = _N_USERS // _B

    def ix(i):
        return (i, 0)

    bs = pl.BlockSpec((_B, _D), ix)
    bu_lo = pl.BlockSpec((1, _B, _HD), lambda i: (0, i, 0))
    bu_hi = pl.BlockSpec((1, _B, _HD), lambda i: (1, i, 0))
    bi_lo = pl.BlockSpec((1, _B, _HD), lambda i: (0, i + nb, 0))
    bi_hi = pl.BlockSpec((1, _B, _HD), lambda i: (1, i + nb, 0))
    return pl.pallas_call(
        _epi_body,
        grid=(nb,),
        in_specs=[bs, bs, bu_lo, bu_hi, bi_lo, bi_hi,
                  bu_lo, bu_hi, bi_lo, bi_hi, bs, bs],
        out_specs=[bs, bs],
        out_shape=[
            jax.ShapeDtypeStruct((_N_USERS, _D), jnp.float32),
            jax.ShapeDtypeStruct((_N_ITEMS, _D), jnp.float32),
        ],
    )(uw, iw, o1, o1, o1, o1, o2, o2, o2, o2, uu, ii)


def kernel(ui_index, ui_values, i2i_index, i2i_values, u2u_index, u2u_values,
           user_ui_w, item_ui_w, uu_w, ii_w):
    zeros_ui = jnp.zeros((_ACC_UI // _NT, _HD), jnp.float32)
    zeros_g = jnp.zeros((_ACC_G // _NT, _D), jnp.float32)

    row_c = jnp.concatenate([_pad1(i2i_index[0], _E_G_PAD, _N_ITEMS),
                             _pad1(u2u_index[0], _E_G_PAD, _N_USERS)])
    col_c = jnp.concatenate([_pad1(i2i_index[1], _E_G_PAD, 0),
                             _pad1(u2u_index[1], _E_G_PAD, 0)])
    val_c = jnp.concatenate([_pad1(i2i_values, _E_G_PAD, 0.0),
                             _pad1(u2u_values, _E_G_PAD, 0.0)])
    x_c = jnp.concatenate([ii_w, uu_w], axis=0)
    out_c = _spmm_cmb(row_c, col_c, val_c, x_c, zeros_g)
    ii_emb = out_c[0, :_N_ITEMS]
    uu_emb = out_c[1, :_N_USERS]

    row_ui = _pad1(ui_index[0], _E_UI_PAD, 0)
    col_ui = _pad1(ui_index[1], _E_UI_PAD, 0)
    val_ui = _pad1(ui_values, _E_UI_PAD, 0.0)
    ego0 = jnp.concatenate([user_ui_w, item_ui_w], axis=0)
    xs0 = jnp.concatenate([ego0[:, :_HD], ego0[:, _HD:]], axis=0)
    o1 = _spmm_ui(row_ui, col_ui, val_ui, xs0, zeros_ui)
    o2 = _spmm_ui(row_ui, col_ui, val_ui, o1.reshape(2 * _N_JOINT, _HD),
                  zeros_ui)

    u_ui_emb, i_ui_emb = _epilogue(user_ui_w, item_ui_w, o1, o2, uu_emb, ii_emb)
    return (u_ui_emb, i_ui_emb, ii_emb, uu_emb)


# speedup vs baseline: 7.9724x; 1.0281x over previous
import functools

import jax
import jax.numpy as jnp
from jax import lax
from jax.experimental import pallas as pl
from jax.experimental.pallas import tpu as pltpu
from jax.experimental.pallas import tpu_sc as plsc

_N_USERS = 25000
_N_ITEMS = 25000
_D = 64
_HD = 32
_N_JOINT = _N_USERS + _N_ITEMS
_NT = 16
_L = 16
def _make_spmm(e_sc: int, per_sc_split: bool, col_off_unit: int,
               d: int, acc_rows: int, c: int, nb: int, dpl: int, dpg: int,
               bf16_gather: bool = False):
    ept = e_sc // _NT
    chunks = ept // c
    assert chunks % nb == 0 and chunks >= nb
    rpt = acc_rows // _NT
    mesh = plsc.VectorSubcoreMesh(core_axis_name="c", subcore_axis_name="s")

    def body(row_h, col_h, val_h, x_h, zeros_h, out_h, acc, pk, vals, idr, idc,
             rows, rowsf, sp, sg, ss):
        cid = lax.axis_index("c")
        sid = lax.axis_index("s")
        col_off = cid * col_off_unit
        ci0 = sid * chunks + (cid * _NT * chunks if per_sc_split else 0)

        def load(ci, b):
            e0 = ci * c
            pltpu.async_copy(row_h.at[pl.ds(e0, c)], pk.at[b, 0], sp.at[b])
            pltpu.async_copy(col_h.at[pl.ds(e0, c)], pk.at[b, 1], sp.at[b])
            pltpu.async_copy(val_h.at[pl.ds(e0, c)], vals.at[b], sp.at[b])

        def gather_prep(ci, b):
            e0 = ci * c
            pltpu.make_async_copy(row_h.at[pl.ds(e0, c)], pk.at[b, 0], sp.at[b]).wait()
            pltpu.make_async_copy(col_h.at[pl.ds(e0, c)], pk.at[b, 1], sp.at[b]).wait()
            pltpu.make_async_copy(val_h.at[pl.ds(e0, c)], vals.at[b], sp.at[b]).wait()
            for g in range(c // _L):
                sl = pl.ds(g * _L, _L)
                idr.at[b][sl] = pk[b, 0, sl]
                idc.at[b][sl] = pk[b, 1, sl] + col_off
            pltpu.async_copy(x_h.at[idc.at[b]], rows.at[b], sg.at[b])

        def wait_g(b):
            pltpu.make_async_copy(x_h.at[idc.at[b]], rows.at[b], sg.at[b]).wait()

        sc_src = rowsf if bf16_gather else rows

        def start_s(b):
            pltpu.async_copy(sc_src.at[b], acc.at[idr.at[b]], ss.at[b], add=True)

        def wait_s(b):
            pltpu.make_async_copy(sc_src.at[b], acc.at[idr.at[b]], ss.at[b]).wait()

        def scale(b):
            rv = rows.at[b]
            rf = rowsf.at[b]
            for g in range(c // _L):
                v16 = vals[b, pl.ds(g * _L, _L)]
                for e in range(_L):
                    s = lax.squeeze(lax.slice(v16, (e,), (e + 1,)), dimensions=(0,))
                    vb = jnp.broadcast_to(s, (_L,))
                    r = g * _L + e
                    if bf16_gather:
                        for k in range(d // (2 * _L)):
                            w16 = rv[r, pl.ds(k * _L, _L)]
                            lo = lax.bitcast_convert_type(w16 << 16, jnp.float32)
                            hi = lax.bitcast_convert_type(
                                w16 & jnp.int32(-65536), jnp.float32)
                            rf[r, pl.ds(k * 2 * _L, _L)] = lo * vb
                            rf[r, pl.ds(k * 2 * _L + _L, _L)] = hi * vb
                    else:
                        for k in range(d // _L):
                            csl = pl.ds(k * _L, _L)
                            rv[r, csl] = rv[r, csl] * vb

        for b in range(dpl):
            load(ci0 + b, b)
        pltpu.sync_copy(zeros_h, acc.at[pl.ds(sid * rpt, rpt)])
        for b in range(dpg):
            gather_prep(ci0 + b, b)
        plsc.subcore_barrier()

        def round_body(j, carry):
            for b in range(nb):
                ii = nb * j + b
                wait_g(b)
                scale(b)
                start_s(b)

                lt = ii + dpl

                @pl.when(lt < chunks)
                def _():
                    load(ci0 + lt, (b + dpl) % nb)

                gt = ii + dpg
                bg = (b + dpg) % nb

                @pl.when(gt < chunks)
                def _():
                    if b < nb - dpg:
                        @pl.when(j > 0)
                        def _():
                            wait_s(bg)
                    else:
                        wait_s(bg)
                    gather_prep(ci0 + gt, bg)

            return carry

        lax.fori_loop(0, chunks // nb, round_body, 0)
        for b in range(nb):
            wait_s(b)
        plsc.subcore_barrier()
        pltpu.sync_copy(acc.at[pl.ds(sid * rpt, rpt)],
                        out_h.at[cid, pl.ds(sid * rpt, rpt)])

    gather_shape = (nb, c, d // 2) if bf16_gather else (nb, c, d)
    gather_dtype = jnp.int32 if bf16_gather else jnp.float32
    rowsf_shape = (nb, c, d) if bf16_gather else (1, _L)
    return pl.kernel(
        body,
        mesh=mesh,
        out_type=jax.ShapeDtypeStruct((2, acc_rows, d), jnp.float32),
        scratch_types=[
            pltpu.VMEM_SHARED((acc_rows, d), jnp.float32),
            pltpu.VMEM((nb, 2, c), jnp.int32),
            pltpu.VMEM((nb, c), jnp.float32),
            pltpu.VMEM((nb, c), jnp.int32),
            pltpu.VMEM((nb, c), jnp.int32),
            pltpu.VMEM(gather_shape, gather_dtype),
            pltpu.VMEM(rowsf_shape, jnp.float32),
            pltpu.SemaphoreType.DMA((nb,)),
            pltpu.SemaphoreType.DMA((nb,)),
            pltpu.SemaphoreType.DMA((nb,)),
        ],
        compiler_params=pltpu.CompilerParams(use_tc_tiling_on_sc=False),
    )


_C_UI = 128
_C_G = 80
_E_UI_PAD = 802816
_E_G_PAD = 403200
_ACC_UI = _N_JOINT
_ACC_G = 25088

_spmm_ui = _make_spmm(_E_UI_PAD, per_sc_split=False, col_off_unit=_N_JOINT,
                      d=_HD, acc_rows=_ACC_UI, c=_C_UI, nb=4, dpl=3, dpg=2,
                      bf16_gather=True)
_spmm_cmb = _make_spmm(_E_G_PAD, per_sc_split=True, col_off_unit=_N_ITEMS,
                       d=_D, acc_rows=_ACC_G, c=_C_G, nb=5, dpl=4, dpg=2)


def _pad1(a, n_pad, fill):
    p = n_pad - a.shape[0]
    return jnp.concatenate([a, jnp.full((p,), fill, a.dtype)]) if p else a


_B = 1000


def _epi_body(uw, iw, e1ul, e1uh, e1il, e1ih, e2ul, e2uh, e2il, e2ih,
              uu, ii, uo, io):
    def l2(x):
        n = jnp.sqrt(jnp.sum(x * x, axis=1, keepdims=True))
        return x / jnp.maximum(n, 1e-12)

    e1u = jnp.concatenate([e1ul[0], e1uh[0]], axis=1)
    e1i = jnp.concatenate([e1il[0], e1ih[0]], axis=1)
    e2u = jnp.concatenate([e2ul[0], e2uh[0]], axis=1)
    e2i = jnp.concatenate([e2il[0], e2ih[0]], axis=1)
    uo[...] = (uw[...] + e1u + e2u) / 3.0 + l2(uu[...])
    io[...] = (iw[...] + e1i + e2i) / 3.0 + l2(ii[...])


def _epilogue(uw, iw, o1, o2, uu, ii):
    nb = _N_USERS // _B

    def ix(i):
        return (i, 0)

    bs = pl.BlockSpec((_B, _D), ix)
    bu_lo = pl.BlockSpec((1, _B, _HD), lambda i: (0, i, 0))
    bu_hi = pl.BlockSpec((1, _B, _HD), lambda i: (1, i, 0))
    bi_lo = pl.BlockSpec((1, _B, _HD), lambda i: (0, i + nb, 0))
    bi_hi = pl.BlockSpec((1, _B, _HD), lambda i: (1, i + nb, 0))
    return pl.pallas_call(
        _epi_body,
        grid=(nb,),
        in_specs=[bs, bs, bu_lo, bu_hi, bi_lo, bi_hi,
                  bu_lo, bu_hi, bi_lo, bi_hi, bs, bs],
        out_specs=[bs, bs],
        out_shape=[
            jax.ShapeDtypeStruct((_N_USERS, _D), jnp.float32),
            jax.ShapeDtypeStruct((_N_ITEMS, _D), jnp.float32),
        ],
    )(uw, iw, o1, o1, o1, o1, o2, o2, o2, o2, uu, ii)


def kernel(ui_index, ui_values, i2i_index, i2i_values, u2u_index, u2u_values,
           user_ui_w, item_ui_w, uu_w, ii_w):
    zeros_ui = jnp.zeros((_ACC_UI // _NT, _HD), jnp.float32)
    zeros_g = jnp.zeros((_ACC_G // _NT, _D), jnp.float32)

    row_c = jnp.concatenate([_pad1(i2i_index[0], _E_G_PAD, _N_ITEMS),
                             _pad1(u2u_index[0], _E_G_PAD, _N_USERS)])
    col_c = jnp.concatenate([_pad1(i2i_index[1], _E_G_PAD, 0),
                             _pad1(u2u_index[1], _E_G_PAD, 0)])
    val_c = jnp.concatenate([_pad1(i2i_values, _E_G_PAD, 0.0),
                             _pad1(u2u_values, _E_G_PAD, 0.0)])
    x_c = jnp.concatenate([ii_w, uu_w], axis=0)
    out_c = _spmm_cmb(row_c, col_c, val_c, x_c, zeros_g)
    ii_emb = out_c[0, :_N_ITEMS]
    uu_emb = out_c[1, :_N_USERS]

    row_ui = _pad1(ui_index[0], _E_UI_PAD, 0)
    col_ui = _pad1(ui_index[1], _E_UI_PAD, 0)
    val_ui = _pad1(ui_values, _E_UI_PAD, 0.0)

    def to_packed_bf16(xs):
        n = xs.shape[0]
        pairs = xs.reshape(n, 2, _L).transpose(0, 2, 1).astype(jnp.bfloat16)
        return lax.bitcast_convert_type(pairs, jnp.int32)

    ego0 = jnp.concatenate([user_ui_w, item_ui_w], axis=0)
    xs0 = jnp.concatenate([ego0[:, :_HD], ego0[:, _HD:]], axis=0)
    o1 = _spmm_ui(row_ui, col_ui, val_ui, to_packed_bf16(xs0), zeros_ui)
    o2 = _spmm_ui(row_ui, col_ui, val_ui,
                  to_packed_bf16(o1.reshape(2 * _N_JOINT, _HD)), zeros_ui)

    u_ui_emb, i_ui_emb = _epilogue(user_ui_w, item_ui_w, o1, o2, uu_emb, ii_emb)
    return (u_ui_emb, i_ui_emb, ii_emb, uu_emb)


# speedup vs baseline: 8.5204x; 1.0687x over previous
import functools

import jax
import jax.numpy as jnp
from jax import lax
from jax.experimental import pallas as pl
from jax.experimental.pallas import tpu as pltpu
from jax.experimental.pallas import tpu_sc as plsc

_N_USERS = 25000
_N_ITEMS = 25000
_D = 64
_HD = 32
_N_JOINT = _N_USERS + _N_ITEMS
_NT = 16
_L = 16
def _make_spmm(e_sc: int, per_sc_split: bool, col_off_unit: int,
               d: int, acc_rows: int, c: int, nb: int, dpl: int, dpg: int,
               bf16_gather: bool = False):
    ept = e_sc // _NT
    chunks = ept // c
    assert chunks % nb == 0 and chunks >= nb
    rpt = acc_rows // _NT
    mesh = plsc.VectorSubcoreMesh(core_axis_name="c", subcore_axis_name="s")

    def body(row_h, col_h, val_h, x_h, zeros_h, out_h, acc, pk, vals, idr, idc,
             rows, rowsf, sp, sg, ss):
        cid = lax.axis_index("c")
        sid = lax.axis_index("s")
        col_off = cid * col_off_unit
        ci0 = sid * chunks + (cid * _NT * chunks if per_sc_split else 0)

        def load(ci, b):
            e0 = ci * c
            pltpu.async_copy(row_h.at[pl.ds(e0, c)], pk.at[b, 0], sp.at[b])
            pltpu.async_copy(col_h.at[pl.ds(e0, c)], pk.at[b, 1], sp.at[b])
            pltpu.async_copy(val_h.at[pl.ds(e0, c)], vals.at[b], sp.at[b])

        def gather_prep(ci, b):
            e0 = ci * c
            pltpu.make_async_copy(row_h.at[pl.ds(e0, c)], pk.at[b, 0], sp.at[b]).wait()
            pltpu.make_async_copy(col_h.at[pl.ds(e0, c)], pk.at[b, 1], sp.at[b]).wait()
            pltpu.make_async_copy(val_h.at[pl.ds(e0, c)], vals.at[b], sp.at[b]).wait()
            for g in range(c // _L):
                sl = pl.ds(g * _L, _L)
                idr.at[b][sl] = pk[b, 0, sl]
                idc.at[b][sl] = pk[b, 1, sl] + col_off
            pltpu.async_copy(x_h.at[idc.at[b]], rows.at[b], sg.at[b])

        def wait_g(b):
            pltpu.make_async_copy(x_h.at[idc.at[b]], rows.at[b], sg.at[b]).wait()

        sc_src = rowsf if bf16_gather else rows

        def start_s(b):
            pltpu.async_copy(sc_src.at[b], acc.at[idr.at[b]], ss.at[b], add=True)

        def wait_s(b):
            pltpu.make_async_copy(sc_src.at[b], acc.at[idr.at[b]], ss.at[b]).wait()

        def scale(b):
            rv = rows.at[b]
            rf = rowsf.at[b]
            for g in range(c // _L):
                v16 = vals[b, pl.ds(g * _L, _L)]
                for e in range(_L):
                    s = lax.squeeze(lax.slice(v16, (e,), (e + 1,)), dimensions=(0,))
                    vb = jnp.broadcast_to(s, (_L,))
                    r = g * _L + e
                    if bf16_gather:
                        for k in range(d // (2 * _L)):
                            w16 = rv[r, pl.ds(k * _L, _L)]
                            lo = lax.bitcast_convert_type(w16 << 16, jnp.float32)
                            hi = lax.bitcast_convert_type(
                                w16 & jnp.int32(-65536), jnp.float32)
                            rf[r, pl.ds(k * 2 * _L, _L)] = lo * vb
                            rf[r, pl.ds(k * 2 * _L + _L, _L)] = hi * vb
                    else:
                        for k in range(d // _L):
                            csl = pl.ds(k * _L, _L)
                            rv[r, csl] = rv[r, csl] * vb

        for b in range(dpl):
            load(ci0 + b, b)
        pltpu.sync_copy(zeros_h, acc.at[pl.ds(sid * rpt, rpt)])
        for b in range(dpg):
            gather_prep(ci0 + b, b)
        plsc.subcore_barrier()

        def round_body(j, carry):
            for b in range(nb):
                ii = nb * j + b
                wait_g(b)
                scale(b)
                start_s(b)

                lt = ii + dpl

                @pl.when(lt < chunks)
                def _():
                    load(ci0 + lt, (b + dpl) % nb)

                gt = ii + dpg
                bg = (b + dpg) % nb

                @pl.when(gt < chunks)
                def _():
                    if b < nb - dpg:
                        @pl.when(j > 0)
                        def _():
                            wait_s(bg)
                    else:
                        wait_s(bg)
                    gather_prep(ci0 + gt, bg)

            return carry

        lax.fori_loop(0, chunks // nb, round_body, 0)
        for b in range(nb):
            wait_s(b)
        plsc.subcore_barrier()
        pltpu.sync_copy(acc.at[pl.ds(sid * rpt, rpt)],
                        out_h.at[cid, pl.ds(sid * rpt, rpt)])

    gather_shape = (nb, c, d // 2) if bf16_gather else (nb, c, d)
    gather_dtype = jnp.int32 if bf16_gather else jnp.float32
    rowsf_shape = (nb, c, d) if bf16_gather else (1, _L)
    return pl.kernel(
        body,
        mesh=mesh,
        out_type=jax.ShapeDtypeStruct((2, acc_rows, d), jnp.float32),
        scratch_types=[
            pltpu.VMEM_SHARED((acc_rows, d), jnp.float32),
            pltpu.VMEM((nb, 2, c), jnp.int32),
            pltpu.VMEM((nb, c), jnp.float32),
            pltpu.VMEM((nb, c), jnp.int32),
            pltpu.VMEM((nb, c), jnp.int32),
            pltpu.VMEM(gather_shape, gather_dtype),
            pltpu.VMEM(rowsf_shape, jnp.float32),
            pltpu.SemaphoreType.DMA((nb,)),
            pltpu.SemaphoreType.DMA((nb,)),
            pltpu.SemaphoreType.DMA((nb,)),
        ],
        compiler_params=pltpu.CompilerParams(use_tc_tiling_on_sc=False),
    )


_C_UI = 128
_C_G = 64
_E_UI_PAD = 802816
_E_G_PAD = 401408
_ACC_UI = _N_JOINT
_ACC_G = 25088

_spmm_ui = _make_spmm(_E_UI_PAD, per_sc_split=False, col_off_unit=_N_JOINT,
                      d=_HD, acc_rows=_ACC_UI, c=_C_UI, nb=4, dpl=3, dpg=2,
                      bf16_gather=True)
_spmm_cmb = _make_spmm(_E_G_PAD, per_sc_split=True, col_off_unit=_N_ITEMS,
                       d=_D, acc_rows=_ACC_G, c=_C_G, nb=4, dpl=3, dpg=2,
                       bf16_gather=True)


def _pad1(a, n_pad, fill):
    p = n_pad - a.shape[0]
    return jnp.concatenate([a, jnp.full((p,), fill, a.dtype)]) if p else a


_B = 1000


def _epi_body(uw, iw, e1ul, e1uh, e1il, e1ih, e2ul, e2uh, e2il, e2ih,
              uu, ii, uo, io):
    def l2(x):
        n = jnp.sqrt(jnp.sum(x * x, axis=1, keepdims=True))
        return x / jnp.maximum(n, 1e-12)

    e1u = jnp.concatenate([e1ul[0], e1uh[0]], axis=1)
    e1i = jnp.concatenate([e1il[0], e1ih[0]], axis=1)
    e2u = jnp.concatenate([e2ul[0], e2uh[0]], axis=1)
    e2i = jnp.concatenate([e2il[0], e2ih[0]], axis=1)
    uo[...] = (uw[...] + e1u + e2u) / 3.0 + l2(uu[...])
    io[...] = (iw[...] + e1i + e2i) / 3.0 + l2(ii[...])


def _epilogue(uw, iw, o1, o2, uu, ii):
    nb = _N_USERS // _B

    def ix(i):
        return (i, 0)

    bs = pl.BlockSpec((_B, _D), ix)
    bu_lo = pl.BlockSpec((1, _B, _HD), lambda i: (0, i, 0))
    bu_hi = pl.BlockSpec((1, _B, _HD), lambda i: (1, i, 0))
    bi_lo = pl.BlockSpec((1, _B, _HD), lambda i: (0, i + nb, 0))
    bi_hi = pl.BlockSpec((1, _B, _HD), lambda i: (1, i + nb, 0))
    return pl.pallas_call(
        _epi_body,
        grid=(nb,),
        in_specs=[bs, bs, bu_lo, bu_hi, bi_lo, bi_hi,
                  bu_lo, bu_hi, bi_lo, bi_hi, bs, bs],
        out_specs=[bs, bs],
        out_shape=[
            jax.ShapeDtypeStruct((_N_USERS, _D), jnp.float32),
            jax.ShapeDtypeStruct((_N_ITEMS, _D), jnp.float32),
        ],
    )(uw, iw, o1, o1, o1, o1, o2, o2, o2, o2, uu, ii)


def kernel(ui_index, ui_values, i2i_index, i2i_values, u2u_index, u2u_values,
           user_ui_w, item_ui_w, uu_w, ii_w):
    zeros_ui = jnp.zeros((_ACC_UI // _NT, _HD), jnp.float32)
    zeros_g = jnp.zeros((_ACC_G // _NT, _D), jnp.float32)

    row_c = jnp.concatenate([_pad1(i2i_index[0], _E_G_PAD, _N_ITEMS),
                             _pad1(u2u_index[0], _E_G_PAD, _N_USERS)])
    col_c = jnp.concatenate([_pad1(i2i_index[1], _E_G_PAD, 0),
                             _pad1(u2u_index[1], _E_G_PAD, 0)])
    val_c = jnp.concatenate([_pad1(i2i_values, _E_G_PAD, 0.0),
                             _pad1(u2u_values, _E_G_PAD, 0.0)])
    x_c = jnp.concatenate([ii_w, uu_w], axis=0)
    n = x_c.shape[0]
    xp_c = lax.bitcast_convert_type(
        x_c.reshape(n, 2, 2, _L).transpose(0, 1, 3, 2).astype(jnp.bfloat16),
        jnp.int32).reshape(n, _D // 2)
    out_c = _spmm_cmb(row_c, col_c, val_c, xp_c, zeros_g)
    ii_emb = out_c[0, :_N_ITEMS]
    uu_emb = out_c[1, :_N_USERS]

    row_ui = _pad1(ui_index[0], _E_UI_PAD, 0)
    col_ui = _pad1(ui_index[1], _E_UI_PAD, 0)
    val_ui = _pad1(ui_values, _E_UI_PAD, 0.0)

    def to_packed_bf16(xs):
        n = xs.shape[0]
        pairs = xs.reshape(n, 2, _L).transpose(0, 2, 1).astype(jnp.bfloat16)
        return lax.bitcast_convert_type(pairs, jnp.int32)

    ego0 = jnp.concatenate([user_ui_w, item_ui_w], axis=0)
    xs0 = jnp.concatenate([ego0[:, :_HD], ego0[:, _HD:]], axis=0)
    o1 = _spmm_ui(row_ui, col_ui, val_ui, to_packed_bf16(xs0), zeros_ui)
    o2 = _spmm_ui(row_ui, col_ui, val_ui,
                  to_packed_bf16(o1.reshape(2 * _N_JOINT, _HD)), zeros_ui)

    u_ui_emb, i_ui_emb = _epilogue(user_ui_w, item_ui_w, o1, o2, uu_emb, ii_emb)
    return (u_ui_emb, i_ui_emb, ii_emb, uu_emb)
